# Initial kernel scaffold; baseline (speedup 1.0000x reference)
import jax
import jax.numpy as jnp
from jax.experimental import pallas as pl


def kernel(x_cat, x_num, tables, W1, b1, g1, be1, W2, b2, g2, be2, W3, b3):
    raise NotImplementedError("write your pallas kernel here")


# speedup vs baseline: 8.0146x; 8.0146x over previous
import jax
import jax.numpy as jnp
from jax import lax
from jax.experimental import pallas as pl
from jax.experimental.pallas import tpu as pltpu
from jax.experimental.pallas import tpu_sc as plsc

B = 16384
F = 26
V = 100000
D = 32
NUM = 13
BF = B * F
ED = F * D
EPS = 1e-5

NC = 2
NS = 16
NW = NC * NS
PER_W = BF // NW
CH = 128
NCH = PER_W // CH
NBUF = 4


def _sc_gather_body(tab, idx2, out, idx_v, rows_v, gsem):
    wid = lax.axis_index("s") * NC + lax.axis_index("c")
    pltpu.sync_copy(idx2.at[pl.ds(wid * NCH, NCH)], idx_v)
    row0 = wid * PER_W

    def outer(go, carry):
        g0 = go * NBUF
        for b in range(NBUF):
            pltpu.async_copy(tab.at[idx_v.at[g0 + b]], rows_v.at[b], gsem)
        for b in range(NBUF):
            pltpu.make_async_copy(tab.at[idx_v.at[g0 + b]], rows_v.at[b], gsem).wait()
            pltpu.sync_copy(rows_v.at[b], out.at[pl.ds(row0 + (g0 + b) * CH, CH)])
        return carry

    lax.fori_loop(0, NCH // NBUF, outer, 0)


_SC_GATHER_CACHE = []


def _sc_gather(tab_flat, flat_idx):
    if not _SC_GATHER_CACHE:
        _SC_GATHER_CACHE.append(pl.kernel(
            _sc_gather_body,
            out_type=jax.ShapeDtypeStruct((BF, D), jnp.float32),
            mesh=plsc.VectorSubcoreMesh(core_axis_name="c", subcore_axis_name="s"),
            scratch_types=[
                pltpu.VMEM((NCH, CH), jnp.int32),
                pltpu.VMEM((NBUF, CH, D), jnp.float32),
                pltpu.SemaphoreType.DMA,
            ],
            compiler_params=pltpu.CompilerParams(use_tc_tiling_on_sc=False),
        ))
    return _SC_GATHER_CACHE[0](tab_flat, flat_idx)


BB = 1024
_INV_STD = (1.0 + EPS) ** -0.5


def _mlp_body(xe, xn, w1e, w1n, b1, g1, be1, w2, b2, g2, be2, w3, b3, out):
    h = jnp.dot(xe[...], w1e[...], preferred_element_type=jnp.float32)
    h = h + jnp.dot(xn[...], w1n[...], preferred_element_type=jnp.float32)
    h = jnp.maximum(h + b1[...], 0.0)
    h = h * (g1[...] * _INV_STD) + be1[...]
    h = jnp.maximum(jnp.dot(h, w2[...], preferred_element_type=jnp.float32) + b2[...], 0.0)
    h = h * (g2[...] * _INV_STD) + be2[...]
    out[...] = jnp.dot(h, w3[...], preferred_element_type=jnp.float32) + b3[...]


_mlp = pl.pallas_call(
    _mlp_body,
    grid=(B // BB,),
    in_specs=[
        pl.BlockSpec((BB, ED), lambda i: (i, 0)),
        pl.BlockSpec((BB, NUM), lambda i: (i, 0)),
        pl.BlockSpec((ED, 128), lambda i: (0, 0)),
        pl.BlockSpec((NUM, 128), lambda i: (0, 0)),
        pl.BlockSpec((1, 128), lambda i: (0, 0)),
        pl.BlockSpec((1, 128), lambda i: (0, 0)),
        pl.BlockSpec((1, 128), lambda i: (0, 0)),
        pl.BlockSpec((128, 64), lambda i: (0, 0)),
        pl.BlockSpec((1, 64), lambda i: (0, 0)),
        pl.BlockSpec((1, 64), lambda i: (0, 0)),
        pl.BlockSpec((1, 64), lambda i: (0, 0)),
        pl.BlockSpec((64, 1), lambda i: (0, 0)),
        pl.BlockSpec((1, 1), lambda i: (0, 0)),
    ],
    out_specs=pl.BlockSpec((BB, 1), lambda i: (i, 0)),
    out_shape=jax.ShapeDtypeStruct((B, 1), jnp.float32),
)


def kernel(x_cat, x_num, tables, W1, b1, g1, be1, W2, b2, g2, be2, W3, b3):
    offs = jnp.arange(F, dtype=jnp.int32) * V
    flat_idx = (x_cat + offs[None, :]).reshape(BF // CH, CH)
    tab_flat = tables.reshape(F * V, D)
    emb = _sc_gather(tab_flat, flat_idx)
    xe = emb.reshape(B, ED)
    return _mlp(
        xe, x_num, W1[:ED], W1[ED:],
        b1.reshape(1, 128), g1.reshape(1, 128), be1.reshape(1, 128),
        W2, b2.reshape(1, 64), g2.reshape(1, 64), be2.reshape(1, 64),
        W3, b3.reshape(1, 1),
    )
